# baseline (speedup 1.0000x reference)
import functools

import jax
import jax.numpy as jnp
from jax import lax
from jax.experimental import pallas as pl
from jax.experimental.pallas import tpu as pltpu
from jax.experimental.pallas import tpu_sc as plsc

_EMB = 128
_BATCH = 16384
_VOCAB = 3

_INFO = plsc.get_sparse_core_info()
_NC = _INFO.num_cores
_NS = _INFO.num_subcores
_L = _INFO.num_lanes
_NCHW = _EMB // _L
_GSZ = _L * _EMB

_BPW0 = 416
_BPW1 = 608
_BMAX = max(_BPW0, _BPW1)
_CORE1_BASE = _NS * _BPW0

_mesh = plsc.VectorSubcoreMesh(core_axis_name="c", subcore_axis_name="s")


@functools.partial(
    pl.kernel,
    mesh=_mesh,
    compiler_params=pltpu.CompilerParams(needs_layout_passes=False),
    out_type=jax.ShapeDtypeStruct((_BATCH * _EMB,), jnp.float32),
    scratch_types=[
        pltpu.VMEM((_BMAX,), jnp.int32),
        pltpu.VMEM((_VOCAB * _EMB,), jnp.float32),
        pltpu.VMEM((_BMAX * _EMB,), jnp.float32),
        pltpu.SemaphoreType.DMA,
        pltpu.SemaphoreType.DMA,
    ],
)
def _embed_lookup(idx_hbm, table_hbm, out_hbm, idx_v, table_v, out_v, sem_in, sem_out):
    cid = lax.axis_index("c")
    sid = lax.axis_index("s")
    row0 = jnp.where(cid == 0, sid * _BPW0, _CORE1_BASE + sid * _BPW1)
    ngrp = jnp.where(cid == 0, _BPW0 // _L, _BPW1 // _L)

    cp_tab = pltpu.async_copy(table_hbm, table_v, sem_in)
    cp_idx = pltpu.async_copy(idx_hbm.at[pl.ds(row0, _BMAX)], idx_v, sem_in)
    cp_tab.wait()
    cp_idx.wait()

    r2 = [table_v[pl.ds(2 * _EMB + c * _L, _L)] for c in range(_NCHW)]
    d0 = [table_v[pl.ds(0 * _EMB + c * _L, _L)] - r2[c] for c in range(_NCHW)]
    d1 = [table_v[pl.ds(1 * _EMB + c * _L, _L)] - r2[c] for c in range(_NCHW)]
    out_base = row0 * _EMB

    @plsc.parallel_loop(0, ngrp, unroll=1)
    def _group(g):
        vidx = idx_v[pl.ds(g * _L, _L)]
        gbase = g * _GSZ
        for j in range(_L):
            vb = jnp.take_along_axis(
                vidx, jnp.full((_L,), j, jnp.int32), axis=0,
                mode="promise_in_bounds",
            )
            f0 = (vb == 0).astype(jnp.float32)
            f1 = (vb == 1).astype(jnp.float32)
            base = gbase + j * _EMB
            for c in range(_NCHW):
                out_v[pl.ds(base + c * _L, _L)] = r2[c] + f0 * d0[c] + f1 * d1[c]
        pltpu.async_copy(
            out_v.at[pl.ds(gbase, _GSZ)],
            out_hbm.at[pl.ds(out_base + gbase, _GSZ)],
            sem_out,
        )

    def _drain(g, carry):
        gbase = g * _GSZ
        pltpu.make_async_copy(
            out_hbm.at[pl.ds(out_base + gbase, _GSZ)],
            out_v.at[pl.ds(gbase, _GSZ)],
            sem_out,
        ).wait()
        return carry

    lax.fori_loop(0, ngrp, _drain, 0)


def kernel(x, table):
    idx = x.reshape(_BATCH).astype(jnp.int32)
    flat = _embed_lookup(idx, table.reshape(_VOCAB * _EMB))
    return flat.reshape(_BATCH, _EMB)


# speedup vs baseline: 1.4140x; 1.4140x over previous
import functools

import jax
import jax.numpy as jnp
from jax import lax
from jax.experimental import pallas as pl
from jax.experimental.pallas import tpu as pltpu
from jax.experimental.pallas import tpu_sc as plsc

_EMB = 128
_BATCH = 16384
_VOCAB = 3

_INFO = plsc.get_sparse_core_info()
_NC = _INFO.num_cores
_NS = _INFO.num_subcores
_NW = _NC * _NS
_BPW = _BATCH // _NW
_L = _INFO.num_lanes
_NCHW = _EMB // _L
_GSZ = _L * _EMB
_NGRP = _BPW // _L

_mesh = plsc.VectorSubcoreMesh(core_axis_name="c", subcore_axis_name="s")


@functools.partial(
    pl.kernel,
    mesh=_mesh,
    compiler_params=pltpu.CompilerParams(needs_layout_passes=False),
    out_type=jax.ShapeDtypeStruct((_BATCH * _EMB,), jnp.float32),
    scratch_types=[
        pltpu.VMEM((_BPW,), jnp.int32),
        pltpu.VMEM((_VOCAB * _EMB,), jnp.float32),
        pltpu.VMEM((_BPW * _EMB,), jnp.float32),
        pltpu.SemaphoreType.DMA,
        pltpu.SemaphoreType.DMA,
    ],
)
def _embed_lookup(idx_hbm, table_hbm, out_hbm, idx_v, table_v, out_v, sem_in, sem_out):
    wid = lax.axis_index("s") * _NC + lax.axis_index("c")
    cp_idx = pltpu.async_copy(idx_hbm.at[wid], idx_v, sem_in)
    cp_tab = pltpu.async_copy(table_hbm, table_v, sem_in)
    cp_idx.wait()
    cp_tab.wait()

    rows = [
        [table_v[pl.ds(k * _EMB + c * _L, _L)] for c in range(_NCHW)]
        for k in range(_VOCAB)
    ]
    out_base = wid * (_BPW * _EMB)

    @plsc.parallel_loop(0, _NGRP, unroll=1)
    def _group(g):
        vidx = idx_v[pl.ds(g * _L, _L)]
        gbase = g * _GSZ
        for j in range(_L):
            vb = jnp.take_along_axis(
                vidx, jnp.full((_L,), j, jnp.int32), axis=0,
                mode="promise_in_bounds",
            )
            m0 = vb == 0
            m1 = vb == 1
            base = gbase + j * _EMB
            for c in range(_NCHW):
                val = jnp.where(m0, rows[0][c],
                                jnp.where(m1, rows[1][c], rows[2][c]))
                out_v[pl.ds(base + c * _L, _L)] = val
        pltpu.async_copy(
            out_v.at[pl.ds(gbase, _GSZ)],
            out_hbm.at[pl.ds(out_base + gbase, _GSZ)],
            sem_out,
        )

    pltpu.make_async_copy(
        out_hbm.at[pl.ds(out_base, _BPW * _EMB)], out_v, sem_out
    ).wait()


def kernel(x, table):
    idx = x.reshape(_NW, _BPW).astype(jnp.int32)
    flat = _embed_lookup(idx, table.reshape(_VOCAB * _EMB))
    return flat.reshape(_BATCH, _EMB)
